# baseline (speedup 1.0000x reference)
import jax
import jax.numpy as jnp
from jax.experimental import pallas as pl
from jax.experimental.pallas import tpu as pltpu

_BB = 8


def _body(x_ref, o_ref):
    x = x_ref[...]
    C = x.shape[2]
    iota_s = jax.lax.broadcasted_iota(jnp.int32, (8, C), 0)
    row_c = [iota_s + 8 * k for k in range(4)]
    sent = jnp.full((8, C), 64, jnp.int32)
    out = jnp.zeros((8, C), jnp.int32)
    for b in range(_BB):
        xb = x[b]
        g = [xb[8 * k:8 * (k + 1), :] for k in range(4)]
        t01 = jnp.minimum(g[0], g[1])
        t23 = jnp.minimum(g[2], g[3])
        v = jnp.minimum(t01, t23)
        for sh in (4, 2, 1):
            v = jnp.minimum(v, pltpu.roll(v, sh, axis=0))
        k01 = jnp.where(g[0] == v, row_c[0], row_c[1])
        k23 = jnp.where(g[2] == v, row_c[2], row_c[3])
        km = jnp.where(t23 == v, k23, sent)
        idx = jnp.where(t01 == v, k01, km)
        for sh in (4, 2, 1):
            idx = jnp.minimum(idx, pltpu.roll(idx, sh, axis=0))
        out = jnp.where(iota_s == b, idx, out)
    o_ref[...] = out


def kernel(x):
    B, R, C = x.shape
    return pl.pallas_call(
        _body,
        grid=(B // _BB,),
        in_specs=[pl.BlockSpec((_BB, R, C), lambda i: (i, 0, 0))],
        out_specs=pl.BlockSpec((_BB, C), lambda i: (i, 0)),
        out_shape=jax.ShapeDtypeStruct((B, C), jnp.int32),
    )(x)


# speedup vs baseline: 1.0397x; 1.0397x over previous
import jax
import jax.numpy as jnp
from jax.experimental import pallas as pl
from jax.experimental.pallas import tpu as pltpu

_BB = 16


def _body(x_ref, o_ref):
    x = x_ref[...]
    C = x.shape[2]
    iota_s = jax.lax.broadcasted_iota(jnp.int32, (8, C), 0)
    row_c = [iota_s + 8 * k for k in range(4)]
    sent = jnp.full((8, C), 64, jnp.int32)
    outs = [jnp.zeros((8, C), jnp.int32) for _ in range(_BB // 8)]
    for b in range(_BB):
        xb = x[b]
        g = [xb[8 * k:8 * (k + 1), :] for k in range(4)]
        t01 = jnp.minimum(g[0], g[1])
        t23 = jnp.minimum(g[2], g[3])
        v = jnp.minimum(t01, t23)
        for sh in (4, 2, 1):
            v = jnp.minimum(v, pltpu.roll(v, sh, axis=0))
        k01 = jnp.where(g[0] == v, row_c[0], row_c[1])
        k23 = jnp.where(g[2] == v, row_c[2], row_c[3])
        km = jnp.where(t23 == v, k23, sent)
        idx = jnp.where(t01 == v, k01, km)
        for sh in (4, 2, 1):
            idx = jnp.minimum(idx, pltpu.roll(idx, sh, axis=0))
        outs[b // 8] = jnp.where(iota_s == b % 8, idx, outs[b // 8])
    for j, o in enumerate(outs):
        o_ref[8 * j:8 * (j + 1), :] = o


def kernel(x):
    B, R, C = x.shape
    return pl.pallas_call(
        _body,
        grid=(B // _BB,),
        in_specs=[pl.BlockSpec((_BB, R, C), lambda i: (i, 0, 0))],
        out_specs=pl.BlockSpec((_BB, C), lambda i: (i, 0)),
        out_shape=jax.ShapeDtypeStruct((B, C), jnp.int32),
    )(x)


# speedup vs baseline: 1.0481x; 1.0081x over previous
import jax
import jax.numpy as jnp
from jax.experimental import pallas as pl
from jax.experimental.pallas import tpu as pltpu

_BB = 16


def _body(x_ref, o_ref):
    x = x_ref[...]
    C = x.shape[2]
    iota_i = jax.lax.broadcasted_iota(jnp.int32, (8, C), 0)
    iota_f = iota_i.astype(jnp.float32)
    row_c = [iota_f + 8.0 * k for k in range(4)]
    sent = jnp.full((8, C), 64.0, jnp.float32)
    outs = [jnp.zeros((8, C), jnp.float32) for _ in range(_BB // 8)]
    for b in range(_BB):
        xb = x[b]
        g = [xb[8 * k:8 * (k + 1), :] for k in range(4)]
        t01 = jnp.minimum(g[0], g[1])
        t23 = jnp.minimum(g[2], g[3])
        v = jnp.minimum(t01, t23)
        for sh in (4, 2, 1):
            v = jnp.minimum(v, pltpu.roll(v, sh, axis=0))
        k01 = jnp.where(g[0] == v, row_c[0], row_c[1])
        k23 = jnp.where(g[2] == v, row_c[2], row_c[3])
        km = jnp.where(t23 == v, k23, sent)
        idx = jnp.where(t01 == v, k01, km)
        for sh in (4, 2, 1):
            idx = jnp.minimum(idx, pltpu.roll(idx, sh, axis=0))
        outs[b // 8] = jnp.where(iota_i == b % 8, idx, outs[b // 8])
    for j, o in enumerate(outs):
        o_ref[8 * j:8 * (j + 1), :] = o.astype(jnp.int32)


def kernel(x):
    B, R, C = x.shape
    return pl.pallas_call(
        _body,
        grid=(B // _BB,),
        in_specs=[pl.BlockSpec((_BB, R, C), lambda i: (i, 0, 0))],
        out_specs=pl.BlockSpec((_BB, C), lambda i: (i, 0)),
        out_shape=jax.ShapeDtypeStruct((B, C), jnp.int32),
    )(x)
